# baseline (speedup 1.0000x reference)
import functools

import jax
import jax.numpy as jnp
from jax import lax
from jax.experimental import pallas as pl
from jax.experimental.pallas import tpu as pltpu
from jax.experimental.pallas import tpu_sc as plsc

_BLK = 128
_SPG = 2


@functools.lru_cache(maxsize=None)
def _make_gather(n_fields, n_batch, v, d):
    info = plsc.get_sparse_core_info()
    nw = info.num_cores * info.num_subcores
    nc = info.num_cores
    blocks_per_f = n_batch // _BLK
    n_streams = n_fields * blocks_per_f
    assert n_streams % (nw * _SPG) == 0
    steps_per_w = n_streams // (nw * _SPG)
    assert steps_per_w >= 4 and steps_per_w % 2 == 0

    mesh = plsc.VectorSubcoreMesh(core_axis_name="c", subcore_axis_name="s")

    @functools.partial(
        pl.kernel,
        mesh=mesh,
        out_type=jax.ShapeDtypeStruct((n_fields, d, n_batch), jnp.float32),
        compiler_params=pltpu.CompilerParams(
            use_tc_tiling_on_sc=False, needs_layout_passes=False),
        scratch_types=[
            pltpu.VMEM((2, _SPG, _BLK), jnp.int32),
            pltpu.VMEM((2, _SPG, _BLK, d), jnp.float32),
            pltpu.VMEM((2, _SPG, d, _BLK), jnp.float32),
            pltpu.SemaphoreType.DMA,
            pltpu.SemaphoreType.DMA,
            pltpu.SemaphoreType.DMA,
            pltpu.SemaphoreType.DMA,
            pltpu.SemaphoreType.DMA,
            pltpu.SemaphoreType.DMA,
        ],
    )
    def gather_kernel(table_hbm, idx_hbm, out_hbm, idx_v, rows_v, ct_v,
                      isem0, isem1, gsem0, gsem1, osem0, osem1):
        isem = (isem0, isem1)
        gsem = (gsem0, gsem1)
        osem = (osem0, osem1)
        wid = lax.axis_index("s") * nc + lax.axis_index("c")
        s0 = wid * steps_per_w * _SPG
        lanes = lax.iota(jnp.int32, 16)

        def fb(t, j):
            s = s0 + t * _SPG + j
            return s // blocks_per_f, (s % blocks_per_f) * _BLK

        def fire_idx(t, p):
            for j in range(_SPG):
                f, b0 = fb(t, j)
                pltpu.async_copy(
                    idx_hbm.at[f, pl.ds(b0, _BLK)], idx_v.at[p, j], isem[p])

        def wait_idx(p):
            for j in range(_SPG):
                pltpu.make_async_copy(
                    idx_hbm.at[0, pl.ds(0, _BLK)], idx_v.at[p, j],
                    isem[p]).wait()

        def fire_gather(t, p):
            for j in range(_SPG):
                f, _ = fb(t, j)
                base = f * v

                def rb_body(rb, carry):
                    iv = idx_v[p, j, pl.ds(rb * 16, 16)]
                    for l in range(16):
                        r = base + iv[l]
                        pltpu.async_copy(
                            table_hbm.at[pl.ds(r, 1)],
                            rows_v.at[p, j, pl.ds(rb * 16 + l, 1)], gsem[p])
                    return carry

                lax.fori_loop(0, _BLK // 16, rb_body, 0)

        def wait_gather(p):
            for j in range(_SPG):
                def w_body(n, carry):
                    pltpu.make_async_copy(
                        table_hbm.at[pl.ds(0, 1)],
                        rows_v.at[p, j, pl.ds(0, 1)], gsem[p]).wait()
                    return carry

                lax.fori_loop(0, _BLK, w_body, 0)

        def transpose(p):
            for j in range(_SPG):
                rblk = rows_v.at[p, j]

                def rb_body(rb, carry):
                    rows = rb * 16 + lanes
                    for t in range(d):
                        w = plsc.load_gather(rblk, [rows, lanes * 0 + t])
                        ct_v[p, j, t, pl.ds(rb * 16, 16)] = w
                    return carry

                lax.fori_loop(0, _BLK // 16, rb_body, 0)

        def fire_write(t, p):
            for j in range(_SPG):
                f, b0 = fb(t, j)
                pltpu.async_copy(
                    ct_v.at[p, j], out_hbm.at[f, :, pl.ds(b0, _BLK)], osem[p])

        def wait_write(p):
            for j in range(_SPG):
                pltpu.make_async_copy(
                    ct_v.at[p, j], out_hbm.at[0, :, pl.ds(0, _BLK)],
                    osem[p]).wait()

        fire_idx(0, 0)
        wait_idx(0)
        fire_gather(0, 0)
        fire_idx(1, 1)

        wait_idx(1)
        fire_gather(1, 1)
        fire_idx(2, 0)
        wait_gather(0)
        transpose(0)
        fire_write(0, 0)

        wait_idx(0)
        fire_gather(2, 0)
        fire_idx(3, 1)
        wait_gather(1)
        transpose(1)
        fire_write(1, 1)

        wait_idx(1)
        fire_gather(3, 1)
        fire_idx(4, 0)
        wait_gather(0)
        wait_write(0)
        transpose(0)
        fire_write(2, 0)

        def body(u, carry):
            t0 = 2 * u
            wait_idx(0)
            fire_gather(t0, 0)
            fire_idx(t0 + 1, 1)
            wait_gather(1)
            wait_write(1)
            transpose(1)
            fire_write(t0 - 1, 1)

            wait_idx(1)
            fire_gather(t0 + 1, 1)
            fire_idx((t0 + 2) % steps_per_w, 0)
            wait_gather(0)
            wait_write(0)
            transpose(0)
            fire_write(t0, 0)
            return carry

        lax.fori_loop(2, steps_per_w // 2, body, 0)

        wait_gather(1)
        wait_write(1)
        transpose(1)
        fire_write(steps_per_w - 1, 1)
        wait_write(0)
        wait_write(1)
        wait_idx(0)

    return gather_kernel


def kernel(indices, tables):
    f, v, d = tables.shape
    b = indices.shape[0]
    table2d = tables.reshape(f * v, d)
    idx_t = jnp.transpose(indices.astype(jnp.int32), (1, 0))
    out3 = _make_gather(f, b, v, d)(table2d, idx_t)
    return jnp.transpose(out3, (2, 0, 1))


# speedup vs baseline: 1.0424x; 1.0424x over previous
import functools

import jax
import jax.numpy as jnp
from jax import lax
from jax.experimental import pallas as pl
from jax.experimental.pallas import tpu as pltpu
from jax.experimental.pallas import tpu_sc as plsc

_BLK = 128
_SPG = 2


@functools.lru_cache(maxsize=None)
def _make_gather(n_fields, n_batch, v, d):
    info = plsc.get_sparse_core_info()
    nw = info.num_cores * info.num_subcores
    nc = info.num_cores
    blocks_per_f = n_batch // _BLK
    n_streams = n_fields * blocks_per_f
    assert n_streams % (nw * _SPG) == 0
    steps_per_w = n_streams // (nw * _SPG)
    assert steps_per_w >= 4 and steps_per_w % 2 == 0

    mesh = plsc.VectorSubcoreMesh(core_axis_name="c", subcore_axis_name="s")

    @functools.partial(
        pl.kernel,
        mesh=mesh,
        out_type=jax.ShapeDtypeStruct((n_fields, d, n_batch), jnp.float32),
        compiler_params=pltpu.CompilerParams(
            use_tc_tiling_on_sc=False, needs_layout_passes=False),
        scratch_types=[
            pltpu.VMEM((2, _SPG, _BLK), jnp.int32),
            pltpu.VMEM((2, _SPG, _BLK), jnp.int32),
            pltpu.VMEM((2, _SPG, _BLK, d), jnp.float32),
            pltpu.VMEM((2, _SPG, d, _BLK), jnp.float32),
            pltpu.SemaphoreType.DMA,
            pltpu.SemaphoreType.DMA,
            pltpu.SemaphoreType.DMA,
            pltpu.SemaphoreType.DMA,
            pltpu.SemaphoreType.DMA,
            pltpu.SemaphoreType.DMA,
        ],
    )
    def gather_kernel(table_hbm, idx_hbm, out_hbm, idx_v, pid_v, rows_v, ct_v,
                      isem0, isem1, gsem0, gsem1, osem0, osem1):
        isem = (isem0, isem1)
        gsem = (gsem0, gsem1)
        osem = (osem0, osem1)
        wid = lax.axis_index("s") * nc + lax.axis_index("c")
        s0 = wid * steps_per_w * _SPG
        lanes = lax.iota(jnp.int32, 16)

        def fb(t, j):
            s = s0 + t * _SPG + j
            return s // blocks_per_f, (s % blocks_per_f) * _BLK

        def fire_idx(t, p):
            for j in range(_SPG):
                f, b0 = fb(t, j)
                pltpu.async_copy(
                    idx_hbm.at[f, pl.ds(b0, _BLK)], idx_v.at[p, j], isem[p])

        def wait_idx(p):
            for j in range(_SPG):
                pltpu.make_async_copy(
                    idx_hbm.at[0, pl.ds(0, _BLK)], idx_v.at[p, j],
                    isem[p]).wait()

        def paircomp(t, p):
            for j in range(_SPG):
                f, _ = fb(t, j)
                base = f * v
                for rb in range(_BLK // 16):
                    iv = idx_v[p, j, pl.ds(rb * 16, 16)]
                    pid_v[p, j, pl.ds(rb * 16, 16)] = base + iv

        def fire_gather(p):
            for j in range(_SPG):
                pltpu.async_copy(
                    table_hbm.at[pid_v.at[p, j]], rows_v.at[p, j], gsem[p])

        def wait_gather(p):
            for j in range(_SPG):
                pltpu.make_async_copy(
                    table_hbm.at[pl.ds(0, _BLK)], rows_v.at[p, j],
                    gsem[p]).wait()

        def transpose(p):
            for j in range(_SPG):
                rblk = rows_v.at[p, j]

                def rb_body(rb, carry):
                    rows = rb * 16 + lanes
                    for t in range(d):
                        w = plsc.load_gather(rblk, [rows, lanes * 0 + t])
                        ct_v[p, j, t, pl.ds(rb * 16, 16)] = w
                    return carry

                lax.fori_loop(0, _BLK // 16, rb_body, 0)

        def fire_write(t, p):
            for j in range(_SPG):
                f, b0 = fb(t, j)
                pltpu.async_copy(
                    ct_v.at[p, j], out_hbm.at[f, :, pl.ds(b0, _BLK)], osem[p])

        def wait_write(p):
            for j in range(_SPG):
                pltpu.make_async_copy(
                    ct_v.at[p, j], out_hbm.at[0, :, pl.ds(0, _BLK)],
                    osem[p]).wait()

        fire_idx(0, 0)
        wait_idx(0)
        paircomp(0, 0)
        fire_gather(0)
        fire_idx(1, 1)

        wait_idx(1)
        paircomp(1, 1)
        fire_gather(1)
        fire_idx(2, 0)
        wait_gather(0)
        transpose(0)
        fire_write(0, 0)

        wait_idx(0)
        paircomp(2, 0)
        fire_gather(0)
        fire_idx(3, 1)
        wait_gather(1)
        transpose(1)
        fire_write(1, 1)

        wait_idx(1)
        paircomp(3, 1)
        fire_gather(1)
        fire_idx(4, 0)
        wait_gather(0)
        wait_write(0)
        transpose(0)
        fire_write(2, 0)

        def body(u, carry):
            t0 = 2 * u
            wait_idx(0)
            paircomp(t0, 0)
            fire_gather(0)
            fire_idx(t0 + 1, 1)
            wait_gather(1)
            wait_write(1)
            transpose(1)
            fire_write(t0 - 1, 1)

            wait_idx(1)
            paircomp(t0 + 1, 1)
            fire_gather(1)
            fire_idx((t0 + 2) % steps_per_w, 0)
            wait_gather(0)
            wait_write(0)
            transpose(0)
            fire_write(t0, 0)
            return carry

        lax.fori_loop(2, steps_per_w // 2, body, 0)

        wait_gather(1)
        wait_write(1)
        transpose(1)
        fire_write(steps_per_w - 1, 1)
        wait_write(0)
        wait_write(1)
        wait_idx(0)

    return gather_kernel


def kernel(indices, tables):
    f, v, d = tables.shape
    b = indices.shape[0]
    table2d = tables.reshape(f * v, d)
    idx_t = jnp.transpose(indices.astype(jnp.int32), (1, 0))
    out3 = _make_gather(f, b, v, d)(table2d, idx_t)
    return jnp.transpose(out3, (2, 0, 1))


# speedup vs baseline: 1.2739x; 1.2222x over previous
import functools

import jax
import jax.numpy as jnp
from jax import lax
from jax.experimental import pallas as pl
from jax.experimental.pallas import tpu as pltpu
from jax.experimental.pallas import tpu_sc as plsc

_GRP = 512


@functools.lru_cache(maxsize=None)
def _make_gather(n_rows, d):
    info = plsc.get_sparse_core_info()
    nw = info.num_cores * info.num_subcores
    nc = info.num_cores
    assert n_rows % (nw * _GRP) == 0
    groups_per_w = n_rows // (nw * _GRP)
    assert groups_per_w % 2 == 0 and groups_per_w >= 4
    n_groups = n_rows // _GRP

    mesh = plsc.VectorSubcoreMesh(core_axis_name="c", subcore_axis_name="s")

    @functools.partial(
        pl.kernel,
        mesh=mesh,
        out_type=jax.ShapeDtypeStruct((n_groups, _GRP, d), jnp.float32),
        compiler_params=pltpu.CompilerParams(use_tc_tiling_on_sc=False),
        scratch_types=[
            pltpu.VMEM((2, _GRP), jnp.int32),
            pltpu.VMEM((2, _GRP, d), jnp.float32),
            pltpu.SemaphoreType.DMA,
            pltpu.SemaphoreType.DMA,
            pltpu.SemaphoreType.DMA,
            pltpu.SemaphoreType.DMA,
            pltpu.SemaphoreType.DMA,
            pltpu.SemaphoreType.DMA,
        ],
    )
    def gather_kernel(table_hbm, idx_hbm, out_hbm, idx_v, rows_v,
                      isem0, isem1, gsem0, gsem1, osem0, osem1):
        isem = (isem0, isem1)
        gsem = (gsem0, gsem1)
        osem = (osem0, osem1)
        wid = lax.axis_index("s") * nc + lax.axis_index("c")
        g0 = wid * groups_per_w

        def fire_idx(grp, p):
            pltpu.async_copy(idx_hbm.at[grp], idx_v.at[p], isem[p])

        def wait_idx(p):
            pltpu.make_async_copy(idx_hbm.at[0], idx_v.at[p], isem[p]).wait()

        def fire_gather(p):
            pltpu.async_copy(table_hbm.at[idx_v.at[p]], rows_v.at[p], gsem[p])

        def wait_gather(p):
            pltpu.make_async_copy(out_hbm.at[0], rows_v.at[p], gsem[p]).wait()

        def fire_write(grp, p):
            pltpu.async_copy(rows_v.at[p], out_hbm.at[grp], osem[p])

        def wait_write(p):
            pltpu.make_async_copy(rows_v.at[p], out_hbm.at[0], osem[p]).wait()

        fire_idx(g0, 0)
        wait_idx(0)
        fire_gather(0)
        fire_idx(g0 + 1, 1)
        wait_idx(1)
        wait_gather(0)
        fire_gather(1)
        fire_write(g0, 0)
        fire_idx(g0 + 2, 0)

        def body(u, carry):
            t0 = 2 * u
            wait_write(0)
            wait_idx(0)
            wait_gather(1)
            fire_gather(0)
            fire_write(g0 + t0 - 1, 1)
            fire_idx(g0 + (t0 + 1) % groups_per_w, 1)
            wait_write(1)
            wait_idx(1)
            wait_gather(0)
            fire_gather(1)
            fire_write(g0 + t0, 0)
            fire_idx(g0 + (t0 + 2) % groups_per_w, 0)
            return carry

        lax.fori_loop(1, groups_per_w // 2, body, 0)

        wait_gather(1)
        fire_write(g0 + groups_per_w - 1, 1)
        wait_write(0)
        wait_write(1)
        wait_idx(0)

    return gather_kernel


def kernel(indices, tables):
    f, v, d = tables.shape
    b = indices.shape[0]
    n_rows = b * f
    flat_idx = (
        indices.astype(jnp.int32) + (jnp.arange(f, dtype=jnp.int32) * v)[None, :]
    ).reshape(n_rows // _GRP, _GRP)
    table2d = tables.reshape(f * v, d)
    out = _make_gather(n_rows, d)(table2d, flat_idx)
    return out.reshape(b, f, d)


# speedup vs baseline: 1.2765x; 1.0020x over previous
import functools

import jax
import jax.numpy as jnp
from jax import lax
from jax.experimental import pallas as pl
from jax.experimental.pallas import tpu as pltpu
from jax.experimental.pallas import tpu_sc as plsc

_GRP = 512


@functools.lru_cache(maxsize=None)
def _make_gather(n_rows, d):
    info = plsc.get_sparse_core_info()
    nw = info.num_cores * info.num_subcores
    nc = info.num_cores
    assert n_rows % (nw * _GRP) == 0
    groups_per_w = n_rows // (nw * _GRP)
    assert groups_per_w % 2 == 0 and groups_per_w >= 4
    n_groups = n_rows // _GRP

    mesh = plsc.VectorSubcoreMesh(core_axis_name="c", subcore_axis_name="s")

    @functools.partial(
        pl.kernel,
        mesh=mesh,
        out_type=jax.ShapeDtypeStruct((n_groups, _GRP, d), jnp.float32),
        compiler_params=pltpu.CompilerParams(use_tc_tiling_on_sc=False),
        scratch_types=[
            pltpu.VMEM((2, _GRP), jnp.int32),
            pltpu.VMEM((2, _GRP, d), jnp.float32),
            pltpu.SemaphoreType.DMA,
            pltpu.SemaphoreType.DMA,
            pltpu.SemaphoreType.DMA,
            pltpu.SemaphoreType.DMA,
            pltpu.SemaphoreType.DMA,
            pltpu.SemaphoreType.DMA,
        ],
    )
    def gather_kernel(table_hbm, idx_hbm, out_hbm, idx_v, rows_v,
                      isem0, isem1, gsem0, gsem1, osem0, osem1):
        isem = (isem0, isem1)
        gsem = (gsem0, gsem1)
        osem = (osem0, osem1)
        wid = lax.axis_index("s") * nc + lax.axis_index("c")
        g0 = wid * groups_per_w

        def fire_idx(grp, p):
            pltpu.async_copy(idx_hbm.at[grp], idx_v.at[p], isem[p])

        def wait_idx(p):
            pltpu.make_async_copy(idx_hbm.at[0], idx_v.at[p], isem[p]).wait()

        def fire_gather(p):
            def rb_body(rb, carry):
                iv = idx_v[p, pl.ds(rb * 16, 16)]
                pltpu.async_copy(
                    table_hbm.at[iv], rows_v.at[p, pl.ds(rb * 16, 16)],
                    gsem[p])
                return carry

            lax.fori_loop(0, _GRP // 16, rb_body, 0)

        def wait_gather(p):
            def w_body(n, carry):
                pltpu.make_async_copy(
                    table_hbm.at[pl.ds(0, 16)],
                    rows_v.at[p, pl.ds(0, 16)], gsem[p]).wait()
                return carry

            lax.fori_loop(0, _GRP // 16, w_body, 0)

        def fire_write(grp, p):
            pltpu.async_copy(rows_v.at[p], out_hbm.at[grp], osem[p])

        def wait_write(p):
            pltpu.make_async_copy(rows_v.at[p], out_hbm.at[0], osem[p]).wait()

        fire_idx(g0, 0)
        wait_idx(0)
        fire_gather(0)
        fire_idx(g0 + 1, 1)
        wait_idx(1)
        wait_gather(0)
        fire_gather(1)
        fire_write(g0, 0)
        fire_idx(g0 + 2, 0)

        def body(u, carry):
            t0 = 2 * u
            wait_write(0)
            wait_idx(0)
            wait_gather(1)
            fire_gather(0)
            fire_write(g0 + t0 - 1, 1)
            fire_idx(g0 + (t0 + 1) % groups_per_w, 1)
            wait_write(1)
            wait_idx(1)
            wait_gather(0)
            fire_gather(1)
            fire_write(g0 + t0, 0)
            fire_idx(g0 + (t0 + 2) % groups_per_w, 0)
            return carry

        lax.fori_loop(1, groups_per_w // 2, body, 0)

        wait_gather(1)
        fire_write(g0 + groups_per_w - 1, 1)
        wait_write(0)
        wait_write(1)
        wait_idx(0)

    return gather_kernel


def kernel(indices, tables):
    f, v, d = tables.shape
    b = indices.shape[0]
    n_rows = b * f
    flat_idx = (
        indices.astype(jnp.int32) + (jnp.arange(f, dtype=jnp.int32) * v)[None, :]
    ).reshape(n_rows // _GRP, _GRP)
    table2d = tables.reshape(f * v, d)
    out = _make_gather(n_rows, d)(table2d, flat_idx)
    return out.reshape(b, f, d)
